# baseline (speedup 1.0000x reference)
import functools

import jax
import jax.numpy as jnp
from jax import lax
from jax.experimental import pallas as pl
from jax.experimental.pallas import tpu as pltpu
from jax.experimental.pallas import tpu_sc as plsc

VOCAB = 100000
EMBED = 128
MAXLEN = 200
NSEQ = 4096
PAD = 0

NC = 2
NS = 16
NW = NC * NS
SEQ_PER_W = NSEQ // NW
HALF = MAXLEN // 2
HPW = 2 * SEQ_PER_W
NSLOT = 6
GAHEAD = 3
LANES = 16


def _sc_embed(seq3, table, pe2):
    mesh = plsc.VectorSubcoreMesh(
        core_axis_name="c", subcore_axis_name="s", num_cores=NC, num_subcores=NS
    )

    @functools.partial(
        pl.kernel,
        out_type=jax.ShapeDtypeStruct((2 * NSEQ, HALF, EMBED), jnp.float32),
        mesh=mesh,
        scratch_types=[
            pltpu.VMEM((3, 2, HALF), jnp.int32),
            pltpu.VMEM((NSLOT, HALF, EMBED), jnp.float32),
            pltpu.VMEM((MAXLEN, EMBED), jnp.float32),
            [pltpu.SemaphoreType.DMA] * NSLOT,
            [pltpu.SemaphoreType.DMA] * NSLOT,
            [pltpu.SemaphoreType.DMA] * 3,
        ],
    )
    def body(seq_hbm, table_hbm, pe_hbm, out_hbm, idx_v, rows_v, pe_v,
             gsems, osems, isems):
        wid = lax.axis_index("s") * NC + lax.axis_index("c")
        base = wid * SEQ_PER_W
        base2 = wid * HPW
        pltpu.sync_copy(pe_hbm, pe_v)

        def fire_idx(q, i):
            pltpu.async_copy(seq_hbm.at[base + i], idx_v.at[q], isems[q])

        def wait_idx(q, i):
            pltpu.make_async_copy(
                seq_hbm.at[base + i], idx_v.at[q], isems[q]
            ).wait()

        def fire_gather(slot, h, par):
            del h
            pltpu.async_copy(
                table_hbm.at[idx_v.at[slot // 2, par]], rows_v.at[slot], gsems[slot]
            )

        def wait_gather(slot, par):
            pltpu.make_async_copy(
                table_hbm.at[idx_v.at[slot // 2, par]], rows_v.at[slot], gsems[slot]
            ).wait()

        def add_pe(slot, par):
            @plsc.parallel_loop(0, HALF, step=1, unroll=4)
            def _(r):
                for cc in range(EMBED // LANES):
                    sl = pl.ds(cc * LANES, LANES)
                    plsc.addupdate(rows_v.at[slot, r, sl], pe_v[par * HALF + r, sl])

        def fire_store(slot, h):
            pltpu.async_copy(rows_v.at[slot], out_hbm.at[base2 + h], osems[slot])

        def wait_store(slot, h):
            pltpu.make_async_copy(
                rows_v.at[slot], out_hbm.at[base2 + h], osems[slot]
            ).wait()

        pltpu.sync_copy(seq_hbm.at[base], idx_v.at[0])
        pltpu.sync_copy(seq_hbm.at[base + 1], idx_v.at[1])
        pltpu.sync_copy(seq_hbm.at[base + 2], idx_v.at[2])
        fire_gather(0, 0, 0)
        fire_gather(1, 1, 1)
        fire_gather(2, 2, 0)

        def step(h, b, g, in_loop):
            par = b % 2
            b3 = (b + GAHEAD) % NSLOT
            par3 = (b + GAHEAD) % 2
            wait_gather(b, par)
            if par == 1:
                @pl.when(h + NSLOT < HPW)
                def _pf():
                    fire_idx((b // 2), (h + NSLOT) // 2)
            add_pe(b, par)
            if in_loop or h + GAHEAD < HPW:
                if b < GAHEAD and in_loop:
                    @pl.when(g > 0)
                    def _w():
                        wait_store(b3, h - GAHEAD)
                        if par3 == 0:
                            wait_idx(b3 // 2, (h + GAHEAD) // 2)
                else:
                    wait_store(b3, h - GAHEAD)
                    if par3 == 0 and in_loop:
                        wait_idx(b3 // 2, (h + GAHEAD) // 2)
                fire_gather(b3, h + GAHEAD, par3)
            fire_store(b, h)

        @pl.loop(0, HPW - 4, step=NSLOT)
        def _(g):
            for b in range(NSLOT):
                step(g + b, b, g, True)

        for b in range(4):
            step(HPW - 4 + b, b, None, False)
        wait_store(4, HPW - 6)
        wait_store(5, HPW - 5)
        for b in range(4):
            wait_store(b, HPW - 4 + b)

    return body(seq3, table, pe2)


def _mask_body(x_ref, o_ref):
    o_ref[...] = x_ref[...] == PAD


_mask_call = pl.pallas_call(
    _mask_body,
    out_shape=jax.ShapeDtypeStruct((NSEQ, MAXLEN), jnp.bool_),
    grid=(16,),
    in_specs=[pl.BlockSpec((NSEQ // 16, MAXLEN), lambda i: (i, 0))],
    out_specs=pl.BlockSpec((NSEQ // 16, MAXLEN), lambda i: (i, 0)),
)


@jax.jit
def kernel(input_seq, word_embedding, pe):
    seq = input_seq.astype(jnp.int32)
    seq3 = seq.reshape(NSEQ, 2, HALF)
    pe2 = pe.reshape(MAXLEN, EMBED)
    in_embed = _sc_embed(seq3, word_embedding, pe2).reshape(NSEQ, MAXLEN, EMBED)
    mask = _mask_call(seq)
    return in_embed, mask


# speedup vs baseline: 2.1465x; 2.1465x over previous
import functools

import jax
import jax.numpy as jnp
from jax import lax
from jax.experimental import pallas as pl
from jax.experimental.pallas import tpu as pltpu
from jax.experimental.pallas import tpu_sc as plsc

VOCAB = 100000
EMBED = 128
MAXLEN = 200
NSEQ = 4096
PAD = 0

NC = 2
NS = 16
NW = NC * NS
SEQ_PER_W = NSEQ // NW
HALF = MAXLEN // 2
LANES = 16


def _sc_embed(seq3, table, pe2):
    mesh = plsc.VectorSubcoreMesh(
        core_axis_name="c", subcore_axis_name="s", num_cores=NC, num_subcores=NS
    )

    @functools.partial(
        pl.kernel,
        out_type=jax.ShapeDtypeStruct((NSEQ, MAXLEN, EMBED), jnp.float32),
        mesh=mesh,
        scratch_types=[
            pltpu.VMEM((3, 2, HALF), jnp.int32),
            pltpu.VMEM((3, MAXLEN, EMBED), jnp.float32),
            pltpu.VMEM((MAXLEN, EMBED), jnp.float32),
            pltpu.SemaphoreType.DMA,
            pltpu.SemaphoreType.DMA,
            pltpu.SemaphoreType.DMA,
            pltpu.SemaphoreType.DMA,
            pltpu.SemaphoreType.DMA,
            pltpu.SemaphoreType.DMA,
            pltpu.SemaphoreType.DMA,
            pltpu.SemaphoreType.DMA,
            pltpu.SemaphoreType.DMA,
        ],
    )
    def body(seq_hbm, table_hbm, pe_hbm, out_hbm, idx_v, rows_v, pe_v,
             gsem0, gsem1, gsem2, osem0, osem1, osem2, isem0, isem1, isem2):
        gsems = (gsem0, gsem1, gsem2)
        osems = (osem0, osem1, osem2)
        isems = (isem0, isem1, isem2)
        wid = lax.axis_index("s") * NC + lax.axis_index("c")
        base = wid * SEQ_PER_W
        pltpu.sync_copy(pe_hbm, pe_v)

        def fire_idx(slot, i):
            pltpu.async_copy(seq_hbm.at[base + i], idx_v.at[slot], isems[slot])

        def wait_idx(slot, i):
            pltpu.make_async_copy(
                seq_hbm.at[base + i], idx_v.at[slot], isems[slot]
            ).wait()

        def fire_gather(slot, i):
            del i
            pltpu.async_copy(
                table_hbm.at[idx_v.at[slot, 0]], rows_v.at[slot, pl.ds(0, HALF)],
                gsems[slot],
            )
            pltpu.async_copy(
                table_hbm.at[idx_v.at[slot, 1]], rows_v.at[slot, pl.ds(HALF, HALF)],
                gsems[slot],
            )

        def wait_gather(slot, i):
            del i
            pltpu.make_async_copy(
                table_hbm.at[idx_v.at[slot, 0]], rows_v.at[slot, pl.ds(0, HALF)],
                gsems[slot],
            ).wait()
            pltpu.make_async_copy(
                table_hbm.at[idx_v.at[slot, 1]], rows_v.at[slot, pl.ds(HALF, HALF)],
                gsems[slot],
            ).wait()

        SPLITS = ((0, 96), (96, 104))

        def add_pe_half(slot, hh):
            lo, n = SPLITS[hh]
            @plsc.parallel_loop(lo, lo + n, step=1, unroll=4)
            def _(r):
                for cc in range(EMBED // LANES):
                    sl = pl.ds(cc * LANES, LANES)
                    plsc.addupdate(rows_v.at[slot, r, sl], pe_v[r, sl])

        def fire_store_half(slot, s, hh):
            lo, n = SPLITS[hh]
            pltpu.async_copy(
                rows_v.at[slot, pl.ds(lo, n)],
                out_hbm.at[s, pl.ds(lo, n)],
                osems[slot],
            )

        def wait_store(slot, s):
            for hh in range(2):
                lo, n = SPLITS[hh]
                pltpu.make_async_copy(
                    rows_v.at[slot, pl.ds(lo, n)],
                    out_hbm.at[s, pl.ds(lo, n)],
                    osems[slot],
                ).wait()

        pltpu.sync_copy(seq_hbm.at[base], idx_v.at[0])
        pltpu.sync_copy(seq_hbm.at[base + 1], idx_v.at[1])
        pltpu.sync_copy(seq_hbm.at[base + 2], idx_v.at[2])
        fire_gather(0, 0)
        fire_gather(1, 1)

        @pl.loop(0, SEQ_PER_W - 2, step=3)
        def _(g):
            for b in range(3):
                i = g + b
                s = base + i
                b2 = (b + 2) % 3
                wait_gather(b, i)
                @pl.when(i + 3 < SEQ_PER_W)
                def _pf():
                    fire_idx(b, i + 3)
                add_pe_half(b, 0)
                fire_store_half(b, s, 0)
                add_pe_half(b, 1)
                if b == 0:
                    @pl.when(g > 0)
                    def _w():
                        wait_store(b2, s - 1)
                        wait_idx(b2, i + 2)
                else:
                    wait_idx(b2, i + 2)
                    wait_store(b2, s - 1)
                fire_gather(b2, i + 2)
                fire_store_half(b, s, 1)

        i = SEQ_PER_W - 2
        wait_gather(0, i)
        add_pe_half(0, 0)
        fire_store_half(0, base + i, 0)
        add_pe_half(0, 1)
        fire_store_half(0, base + i, 1)
        wait_store(2, base + i - 1)
        wait_gather(1, i + 1)
        add_pe_half(1, 0)
        fire_store_half(1, base + i + 1, 0)
        add_pe_half(1, 1)
        fire_store_half(1, base + i + 1, 1)
        wait_store(0, base + i)
        wait_store(1, base + i + 1)

    return body(seq3, table, pe2)


def _mask_body(x_ref, o_ref):
    o_ref[...] = x_ref[...] == PAD


_mask_call = pl.pallas_call(
    _mask_body,
    out_shape=jax.ShapeDtypeStruct((NSEQ, MAXLEN), jnp.bool_),
    grid=(16,),
    in_specs=[pl.BlockSpec((NSEQ // 16, MAXLEN), lambda i: (i, 0))],
    out_specs=pl.BlockSpec((NSEQ // 16, MAXLEN), lambda i: (i, 0)),
)


@jax.jit
def kernel(input_seq, word_embedding, pe):
    seq = input_seq.astype(jnp.int32)
    seq3 = seq.reshape(NSEQ, 2, HALF)
    pe2 = pe.reshape(MAXLEN, EMBED)
    in_embed = _sc_embed(seq3, word_embedding, pe2)
    mask = _mask_call(seq)
    return in_embed, mask


# speedup vs baseline: 2.1555x; 1.0042x over previous
import functools

import jax
import jax.numpy as jnp
from jax import lax
from jax.experimental import pallas as pl
from jax.experimental.pallas import tpu as pltpu
from jax.experimental.pallas import tpu_sc as plsc

VOCAB = 100000
EMBED = 128
MAXLEN = 200
NSEQ = 4096
PAD = 0

NC = 2
NS = 16
NW = NC * NS
SEQ_PER_W = NSEQ // NW
HALF = MAXLEN // 2
LANES = 16


def _sc_embed(seq3, table, pe2):
    mesh = plsc.VectorSubcoreMesh(
        core_axis_name="c", subcore_axis_name="s", num_cores=NC, num_subcores=NS
    )

    @functools.partial(
        pl.kernel,
        out_type=jax.ShapeDtypeStruct((NSEQ, MAXLEN, EMBED), jnp.float32),
        mesh=mesh,
        scratch_types=[
            pltpu.VMEM((3, 2, HALF), jnp.int32),
            pltpu.VMEM((3, MAXLEN, EMBED), jnp.float32),
            pltpu.VMEM((MAXLEN, EMBED), jnp.float32),
            pltpu.SemaphoreType.DMA,
            pltpu.SemaphoreType.DMA,
            pltpu.SemaphoreType.DMA,
            pltpu.SemaphoreType.DMA,
            pltpu.SemaphoreType.DMA,
            pltpu.SemaphoreType.DMA,
            pltpu.SemaphoreType.DMA,
            pltpu.SemaphoreType.DMA,
            pltpu.SemaphoreType.DMA,
        ],
    )
    def body(seq_hbm, table_hbm, pe_hbm, out_hbm, idx_v, rows_v, pe_v,
             gsem0, gsem1, gsem2, osem0, osem1, osem2, isem0, isem1, isem2):
        gsems = (gsem0, gsem1, gsem2)
        osems = (osem0, osem1, osem2)
        isems = (isem0, isem1, isem2)
        wid = lax.axis_index("s") * NC + lax.axis_index("c")
        base = wid * SEQ_PER_W

        def fire_idx(slot, i):
            pltpu.async_copy(seq_hbm.at[base + i], idx_v.at[slot], isems[slot])

        def wait_idx(slot, i):
            pltpu.make_async_copy(
                seq_hbm.at[base + i], idx_v.at[slot], isems[slot]
            ).wait()

        def fire_gather(slot, i):
            del i
            pltpu.async_copy(
                table_hbm.at[idx_v.at[slot, 0]], rows_v.at[slot, pl.ds(0, HALF)],
                gsems[slot],
            )
            pltpu.async_copy(
                table_hbm.at[idx_v.at[slot, 1]], rows_v.at[slot, pl.ds(HALF, HALF)],
                gsems[slot],
            )

        def wait_gather(slot, i):
            del i
            pltpu.make_async_copy(
                table_hbm.at[idx_v.at[slot, 0]], rows_v.at[slot, pl.ds(0, HALF)],
                gsems[slot],
            ).wait()
            pltpu.make_async_copy(
                table_hbm.at[idx_v.at[slot, 1]], rows_v.at[slot, pl.ds(HALF, HALF)],
                gsems[slot],
            ).wait()

        SPLITS = ((0, 96), (96, 104))

        def add_pe_half(slot, hh):
            lo, n = SPLITS[hh]
            @plsc.parallel_loop(lo, lo + n, step=1, unroll=4)
            def _(r):
                for cc in range(EMBED // LANES):
                    sl = pl.ds(cc * LANES, LANES)
                    plsc.addupdate(rows_v.at[slot, r, sl], pe_v[r, sl])

        def fire_store_half(slot, s, hh):
            lo, n = SPLITS[hh]
            pltpu.async_copy(
                rows_v.at[slot, pl.ds(lo, n)],
                out_hbm.at[s, pl.ds(lo, n)],
                osems[slot],
            )

        def wait_store(slot, s):
            for hh in range(2):
                lo, n = SPLITS[hh]
                pltpu.make_async_copy(
                    rows_v.at[slot, pl.ds(lo, n)],
                    out_hbm.at[s, pl.ds(lo, n)],
                    osems[slot],
                ).wait()

        pltpu.async_copy(pe_hbm, pe_v, osems[0])
        fire_idx(0, 0)
        fire_idx(1, 1)
        fire_idx(2, 2)
        wait_idx(0, 0)
        fire_gather(0, 0)
        wait_idx(1, 1)
        fire_gather(1, 1)
        pltpu.make_async_copy(pe_hbm, pe_v, osems[0]).wait()

        @pl.loop(0, SEQ_PER_W - 2, step=3)
        def _(g):
            for b in range(3):
                i = g + b
                s = base + i
                b2 = (b + 2) % 3
                wait_gather(b, i)
                @pl.when(i + 3 < SEQ_PER_W)
                def _pf():
                    fire_idx(b, i + 3)
                add_pe_half(b, 0)
                fire_store_half(b, s, 0)
                add_pe_half(b, 1)
                wait_idx(b2, i + 2)
                if b == 0:
                    @pl.when(g > 0)
                    def _w():
                        wait_store(b2, s - 1)
                else:
                    wait_store(b2, s - 1)
                fire_gather(b2, i + 2)
                fire_store_half(b, s, 1)

        i = SEQ_PER_W - 2
        wait_gather(0, i)
        add_pe_half(0, 0)
        fire_store_half(0, base + i, 0)
        add_pe_half(0, 1)
        fire_store_half(0, base + i, 1)
        wait_store(2, base + i - 1)
        wait_gather(1, i + 1)
        add_pe_half(1, 0)
        fire_store_half(1, base + i + 1, 0)
        add_pe_half(1, 1)
        fire_store_half(1, base + i + 1, 1)
        wait_store(0, base + i)
        wait_store(1, base + i + 1)

    return body(seq3, table, pe2)


def _mask_body(x_ref, o_ref):
    o_ref[...] = x_ref[...] == PAD


_mask_call = pl.pallas_call(
    _mask_body,
    out_shape=jax.ShapeDtypeStruct((NSEQ, MAXLEN), jnp.bool_),
    grid=(16,),
    in_specs=[pl.BlockSpec((NSEQ // 16, MAXLEN), lambda i: (i, 0))],
    out_specs=pl.BlockSpec((NSEQ // 16, MAXLEN), lambda i: (i, 0)),
)


@jax.jit
def kernel(input_seq, word_embedding, pe):
    seq = input_seq.astype(jnp.int32)
    seq3 = seq.reshape(NSEQ, 2, HALF)
    pe2 = pe.reshape(MAXLEN, EMBED)
    in_embed = _sc_embed(seq3, word_embedding, pe2)
    mask = _mask_call(seq)
    return in_embed, mask
